# baseline (speedup 1.0000x reference)
import functools

import jax
import jax.numpy as jnp
from jax import lax
from jax.experimental import pallas as pl
from jax.experimental.pallas import tpu as pltpu
from jax.experimental.pallas import tpu_sc as plsc

VOCAB_SIZE = 1000000
N_EMBD = 64
BATCH = 4096
SEQ_LEN = 200

NC, NS = 2, 16
NW = NC * NS
ROWS_PW = BATCH // NW
CHUNKS = (104, 96)
NGROUP = ROWS_PW
NBANK = 4

_mesh = plsc.VectorSubcoreMesh(
    core_axis_name="c", subcore_axis_name="s", num_cores=NC, num_subcores=NS)


@functools.partial(
    pl.kernel,
    out_type=jax.ShapeDtypeStruct((BATCH, SEQ_LEN, N_EMBD), jnp.float32),
    mesh=_mesh,
    compiler_params=pltpu.CompilerParams(use_tc_tiling_on_sc=False),
    scratch_types=[
        pltpu.VMEM((ROWS_PW, SEQ_LEN), jnp.int32),
        pltpu.VMEM((NBANK, SEQ_LEN, N_EMBD), jnp.float32),
        pltpu.SemaphoreType.DMA,
        pltpu.SemaphoreType.DMA,
        pltpu.SemaphoreType.DMA,
        pltpu.SemaphoreType.DMA,
        pltpu.SemaphoreType.DMA,
        pltpu.SemaphoreType.DMA,
        pltpu.SemaphoreType.DMA,
        pltpu.SemaphoreType.DMA,
    ],
)
def _embed_sc(idx_hbm, table_hbm, out_hbm, idx_v, rows_v,
              g0, g1, g2, g3, s0, s1, s2, s3):
    gsems = (g0, g1, g2, g3)
    ssems = (s0, s1, s2, s3)
    wid = lax.axis_index("s") * NC + lax.axis_index("c")
    wbase = wid * ROWS_PW
    pltpu.sync_copy(idx_hbm.at[pl.ds(wbase, ROWS_PW)], idx_v)

    def fire_g(g, bank):
        off = 0
        for c in CHUNKS:
            pltpu.async_copy(table_hbm.at[idx_v.at[g, pl.ds(off, c)]],
                             rows_v.at[bank, pl.ds(off, c)],
                             gsems[bank])
            off += c

    def wait_g(bank):
        pltpu.make_async_copy(table_hbm.at[pl.ds(0, SEQ_LEN)],
                              rows_v.at[bank], gsems[bank]).wait()

    def fire_s(g, bank):
        pltpu.async_copy(rows_v.at[bank], out_hbm.at[wbase + g], ssems[bank])

    def wait_s(g, bank):
        pltpu.make_async_copy(rows_v.at[bank], out_hbm.at[wbase + g],
                              ssems[bank]).wait()

    fire_g(0, 0)
    fire_g(1, 1)
    wait_g(0); fire_s(0, 0); fire_g(2, 2)
    wait_g(1); fire_s(1, 1); fire_g(3, 3)

    def body(p, carry):
        for j in range(NBANK):
            g = NBANK * p + 2 + j
            bank = (2 + j) % NBANK
            nxt = j % NBANK
            wait_g(bank)
            fire_s(g, bank)
            wait_s(g - 2, nxt)
            fire_g(g + 2, nxt)
        return carry

    lax.fori_loop(0, (NGROUP - 4) // NBANK, body, 0)

    wait_g(2); fire_s(NGROUP - 2, 2); wait_s(NGROUP - 4, 0)
    wait_g(3); fire_s(NGROUP - 1, 3); wait_s(NGROUP - 3, 1)
    wait_s(NGROUP - 2, 2)
    wait_s(NGROUP - 1, 3)


def kernel(input_ids, weight):
    return _embed_sc(input_ids, weight)
